# Initial kernel scaffold; baseline (speedup 1.0000x reference)
import functools

import jax
import jax.numpy as jnp
from jax import lax
from jax.experimental import pallas as pl
from jax.experimental.pallas import tpu as pltpu
from jax.experimental.pallas import tpu_sc as plsc


def _matvec_body(emb_ref, w_ref, v_ref):
    v_ref[...] = jnp.dot(emb_ref[...], w_ref[...],
                         preferred_element_type=jnp.float32)


def _emb_matvec(emb, W, interpret=False):
    V, D = emb.shape
    BV = 5000
    return pl.pallas_call(
        _matvec_body,
        grid=(V // BV,),
        in_specs=[pl.BlockSpec((BV, D), lambda i: (i, 0)),
                  pl.BlockSpec((D, 1), lambda i: (0, 0))],
        out_specs=pl.BlockSpec((BV, 1), lambda i: (i, 0)),
        out_shape=jax.ShapeDtypeStruct((V, 1), jnp.float32),
        interpret=interpret,
    )(emb, W)


def _segment_wh_jnp(v, tok, opad, onext, NP):
    p = jnp.take(v, tok, axis=0)
    e = jnp.concatenate([jnp.zeros((1,), jnp.float32), jnp.cumsum(p)])
    e1 = jnp.take(e, opad)
    e2 = jnp.take(e, onext)
    cnt = (onext - opad).astype(jnp.float32)
    return (e2 - e1) / jnp.maximum(cnt, 1.0)


_NW = 32


def _sc_scan(v, tok):
    V, = v.shape
    T, = tok.shape
    CH = T // _NW
    mesh = plsc.VectorSubcoreMesh(core_axis_name="c", subcore_axis_name="s")

    @functools.partial(
        pl.kernel, mesh=mesh,
        out_type=[jax.ShapeDtypeStruct((T + 16,), jnp.float32),
                  jax.ShapeDtypeStruct((_NW, 16), jnp.float32)],
        scratch_types=[pltpu.VMEM((V,), jnp.float32),
                       pltpu.VMEM((CH,), jnp.int32),
                       pltpu.VMEM((CH,), jnp.float32),
                       pltpu.VMEM((16,), jnp.float32),
                       pltpu.VMEM((16,), jnp.float32)],
    )
    def k(v_hbm, tok_hbm, e_hbm, tot_hbm, vbuf, ibuf, pbuf, tbuf, zbuf):
        nc = 2
        wid = lax.axis_index("s") * nc + lax.axis_index("c")
        pltpu.sync_copy(v_hbm, vbuf)
        pltpu.sync_copy(tok_hbm.at[pl.ds(wid * CH, CH)], ibuf)

        def body(kk, carry):
            idx = ibuf[pl.ds(kk * 16, 16)]
            p = plsc.load_gather(vbuf, [idx])
            incl = plsc.cumsum(p)
            pbuf[pl.ds(kk * 16, 16)] = (incl - p) + carry
            return carry + jnp.sum(p)

        total = lax.fori_loop(0, CH // 16, body, jnp.float32(0.0))
        pltpu.sync_copy(pbuf, e_hbm.at[pl.ds(wid * CH, CH)])
        tbuf[...] = jnp.full((16,), total, jnp.float32)
        pltpu.sync_copy(tbuf, tot_hbm.at[wid])

        @pl.when(wid == _NW - 1)
        def _():
            zbuf[...] = jnp.zeros((16,), jnp.float32)
            pltpu.sync_copy(zbuf, e_hbm.at[pl.ds(T, 16)])

    return k(v, tok)


def _sc_boundaries(e, base, o3d, n3d, CH):
    NW, K, L = o3d.shape
    mesh = plsc.VectorSubcoreMesh(core_axis_name="c", subcore_axis_name="s")

    @functools.partial(
        pl.kernel, mesh=mesh,
        out_type=jax.ShapeDtypeStruct((NW, K, L), jnp.float32),
        scratch_types=[pltpu.VMEM((K, L), jnp.int32),
                       pltpu.VMEM((K, L), jnp.int32),
                       pltpu.VMEM((K, L), jnp.float32),
                       pltpu.VMEM((K, L), jnp.float32),
                       pltpu.VMEM((K, L), jnp.float32),
                       pltpu.VMEM((40,), jnp.float32),
                       pltpu.SemaphoreType.DMA],
    )
    def k(e_hbm, base_hbm, o_hbm, n_hbm, wh_hbm,
          obuf, nbuf, e1buf, e2buf, whbuf, bbuf, sem):
        nc = 2
        wid = lax.axis_index("s") * nc + lax.axis_index("c")
        pltpu.sync_copy(base_hbm, bbuf)
        pltpu.sync_copy(o_hbm.at[wid], obuf)
        pltpu.sync_copy(n_hbm.at[wid], nbuf)
        for kk in range(K):
            pltpu.async_copy(e_hbm.at[obuf.at[kk]], e1buf.at[kk], sem).wait()
            pltpu.async_copy(e_hbm.at[nbuf.at[kk]], e2buf.at[kk], sem).wait()
        for kk in range(K):
            for g in range(L // 16):
                sl = pl.ds(g * 16, 16)
                o = obuf[kk, sl]
                onx = nbuf[kk, sl]
                b1 = plsc.load_gather(bbuf, [o // CH])
                b2 = plsc.load_gather(bbuf, [onx // CH])
                e1 = e1buf[kk, sl] + b1
                e2 = e2buf[kk, sl] + b2
                cnt = (onx - o).astype(jnp.float32)
                whbuf[kk, sl] = (e2 - e1) / jnp.maximum(cnt, 1.0)
        pltpu.sync_copy(whbuf, wh_hbm.at[wid])

    return k(e, base, o3d, n3d)


def _segment_wh_sc(v, tok, opad, onext, NP):
    T, = tok.shape
    CH = T // _NW
    e, totals = _sc_scan(v, tok)
    base = jnp.concatenate(
        [jnp.zeros((1,), jnp.float32), jnp.cumsum(totals[:, 0])])
    base = jnp.concatenate([base, jnp.zeros((40 - _NW - 1,), jnp.float32)])
    K = NP // (_NW * 128)
    o3d = opad.reshape(_NW, K, 128)
    n3d = onext.reshape(_NW, K, 128)
    wh = _sc_boundaries(e, base, o3d, n3d, CH)
    return wh.reshape(NP)


def _attn_body(a_ref, whr_ref, whc_ref, adj_ref, out_ref,
               P, Q, Pw, Qw, *, BC, NBJ, NN):
    j = pl.program_id(1)
    a0 = a_ref[0]
    a1 = a_ref[1]
    whc_all = whc_ref[0:1, :]
    D = jnp.max(jnp.maximum(whc_all * a1, 0.0))
    whc_j = whc_ref[0:1, pl.ds(j * BC, BC)]
    d = whc_j * a1
    c1 = jnp.exp(d - D)
    c2 = jnp.exp(0.2 * (d - D))
    cw1 = c1 * whc_j
    cw2 = c2 * whc_j
    s = whr_ref[...] * a0
    adj = adj_ref[...]
    pos = d > (-s)
    ap = jnp.where(pos, adj, 0.0)
    an = adj - ap
    pP = jnp.sum(ap * c1, axis=1, keepdims=True)
    pQ = jnp.sum(an * c2, axis=1, keepdims=True)
    pPw = jnp.sum(ap * cw1, axis=1, keepdims=True)
    pQw = jnp.sum(an * cw2, axis=1, keepdims=True)
    first = j == 0
    Pn = jnp.where(first, pP, P[...] + pP)
    Qn = jnp.where(first, pQ, Q[...] + pQ)
    Pwn = jnp.where(first, pPw, Pw[...] + pPw)
    Qwn = jnp.where(first, pQw, Qw[...] + pQw)
    P[...] = Pn
    Q[...] = Qn
    Pw[...] = Pwn
    Qw[...] = Qwn

    @pl.when(j == NBJ - 1)
    def _():
        sD = s + D
        m = jnp.where(sD > 0, sD, 0.2 * sD)
        r1 = jnp.exp(sD - m)
        r2 = jnp.exp(0.2 * sD - m)
        S1 = r1 * Pn + r2 * Qn
        N1 = r1 * Pwn + r2 * Qwn
        meanwh = jnp.sum(whc_all) / NN
        out_ref[...] = jnp.where(S1 > 0, N1 / S1, meanwh)


def _attention(adj, whr, whc, a2, interpret=False):
    NN = adj.shape[0]
    NP = whc.shape[1]
    BR, BC = 1000, 2000
    NBI, NBJ = NN // BR, NN // BC
    body = functools.partial(_attn_body, BC=BC, NBJ=NBJ, NN=NN)
    return pl.pallas_call(
        body,
        grid=(NBI, NBJ),
        in_specs=[pl.BlockSpec(memory_space=pltpu.SMEM),
                  pl.BlockSpec((BR, 1), lambda i, j: (i, 0)),
                  pl.BlockSpec((1, NP), lambda i, j: (0, 0)),
                  pl.BlockSpec((BR, BC), lambda i, j: (i, j))],
        out_specs=pl.BlockSpec((BR, 1), lambda i, j: (i, 0)),
        out_shape=jax.ShapeDtypeStruct((NN, 1), jnp.float32),
        scratch_shapes=[pltpu.VMEM((BR, 1), jnp.float32)] * 4,
        compiler_params=pltpu.CompilerParams(
            dimension_semantics=("arbitrary", "arbitrary")),
        interpret=interpret,
    )(a2, whr, whc, adj)


_USE_SC = False


def kernel(inputs, offsets, adj_matrix, emb_table, W, a):
    T = inputs.shape[0]
    N = offsets.shape[0]
    NP = 12288

    tok = inputs.astype(jnp.int32)
    offs = offsets.astype(jnp.int32)
    opad = jnp.concatenate([offs, jnp.full((NP - N,), T, jnp.int32)])
    onext = jnp.concatenate([offs[1:], jnp.full((NP - N + 1,), T, jnp.int32)])

    v = _emb_matvec(emb_table, W).reshape(-1)
    if _USE_SC:
        wh = _segment_wh_sc(v, tok, opad, onext, NP)
    else:
        wh = _segment_wh_jnp(v, tok, opad, onext, NP)
    whr = wh[:N].reshape(N, 1)
    whc = wh.reshape(1, NP)
    a2 = a.reshape(2).astype(jnp.float32)
    return _attention(adj_matrix, whr, whc, a2)


# speedup vs baseline: 11.5046x; 11.5046x over previous
import functools

import jax
import jax.numpy as jnp
from jax import lax
from jax.experimental import pallas as pl
from jax.experimental.pallas import tpu as pltpu
from jax.experimental.pallas import tpu_sc as plsc


def _matvec_body(emb_ref, w_ref, v_ref):
    v_ref[...] = jnp.dot(emb_ref[...], w_ref[...],
                         preferred_element_type=jnp.float32,
                         precision=lax.Precision.HIGHEST)


def _emb_matvec(emb, W, interpret=False):
    V, D = emb.shape
    BV = 5000
    return pl.pallas_call(
        _matvec_body,
        grid=(V // BV,),
        in_specs=[pl.BlockSpec((BV, D), lambda i: (i, 0)),
                  pl.BlockSpec((D, 1), lambda i: (0, 0))],
        out_specs=pl.BlockSpec((BV, 1), lambda i: (i, 0)),
        out_shape=jax.ShapeDtypeStruct((V, 1), jnp.float32),
        interpret=interpret,
    )(emb, W)


def _segment_wh_jnp(v, tok, opad, onext, NP):
    p = jnp.take(v, tok, axis=0)
    e = jnp.concatenate([jnp.zeros((1,), jnp.float32), jnp.cumsum(p)])
    e1 = jnp.take(e, opad)
    e2 = jnp.take(e, onext)
    cnt = (onext - opad).astype(jnp.float32)
    return (e2 - e1) / jnp.maximum(cnt, 1.0)


_NW = 32


def _sc_scan(v, tok):
    V, = v.shape
    T, = tok.shape
    CH = T // _NW
    mesh = plsc.VectorSubcoreMesh(core_axis_name="c", subcore_axis_name="s")

    @functools.partial(
        pl.kernel, mesh=mesh,
        out_type=[jax.ShapeDtypeStruct((T + 16,), jnp.float32),
                  jax.ShapeDtypeStruct((_NW, 16), jnp.float32)],
        scratch_types=[pltpu.VMEM((V,), jnp.float32),
                       pltpu.VMEM((CH,), jnp.int32),
                       pltpu.VMEM((CH,), jnp.float32),
                       pltpu.VMEM((16,), jnp.float32),
                       pltpu.VMEM((16,), jnp.float32)],
    )
    def k(v_hbm, tok_hbm, e_hbm, tot_hbm, vbuf, ibuf, pbuf, tbuf, zbuf):
        nc = 2
        wid = lax.axis_index("s") * nc + lax.axis_index("c")
        pltpu.sync_copy(v_hbm, vbuf)
        pltpu.sync_copy(tok_hbm.at[pl.ds(wid * CH, CH)], ibuf)

        def body(kk, carry):
            idx = ibuf[pl.ds(kk * 16, 16)]
            p = plsc.load_gather(vbuf, [idx])
            incl = plsc.cumsum(p)
            pbuf[pl.ds(kk * 16, 16)] = (incl - p) + carry
            return carry + jnp.sum(p)

        total = lax.fori_loop(0, CH // 16, body, jnp.float32(0.0))
        pltpu.sync_copy(pbuf, e_hbm.at[pl.ds(wid * CH, CH)])
        tbuf[...] = jnp.full((16,), total, jnp.float32)
        pltpu.sync_copy(tbuf, tot_hbm.at[wid])

        @pl.when(wid == _NW - 1)
        def _():
            zbuf[...] = jnp.zeros((16,), jnp.float32)
            pltpu.sync_copy(zbuf, e_hbm.at[pl.ds(T, 16)])

    return k(v, tok)


def _sc_boundaries(e, base, o3d, n3d, CH):
    NW, K, L = o3d.shape
    mesh = plsc.VectorSubcoreMesh(core_axis_name="c", subcore_axis_name="s")

    @functools.partial(
        pl.kernel, mesh=mesh,
        out_type=jax.ShapeDtypeStruct((NW, K, L), jnp.float32),
        scratch_types=[pltpu.VMEM((K, L), jnp.int32),
                       pltpu.VMEM((K, L), jnp.int32),
                       pltpu.VMEM((K, L), jnp.float32),
                       pltpu.VMEM((K, L), jnp.float32),
                       pltpu.VMEM((K, L), jnp.float32),
                       pltpu.VMEM((40,), jnp.float32),
                       pltpu.SemaphoreType.DMA],
    )
    def k(e_hbm, base_hbm, o_hbm, n_hbm, wh_hbm,
          obuf, nbuf, e1buf, e2buf, whbuf, bbuf, sem):
        nc = 2
        wid = lax.axis_index("s") * nc + lax.axis_index("c")
        pltpu.sync_copy(base_hbm, bbuf)
        pltpu.sync_copy(o_hbm.at[wid], obuf)
        pltpu.sync_copy(n_hbm.at[wid], nbuf)
        for kk in range(K):
            pltpu.async_copy(e_hbm.at[obuf.at[kk]], e1buf.at[kk], sem).wait()
            pltpu.async_copy(e_hbm.at[nbuf.at[kk]], e2buf.at[kk], sem).wait()
        for kk in range(K):
            for g in range(L // 16):
                sl = pl.ds(g * 16, 16)
                o = obuf[kk, sl]
                onx = nbuf[kk, sl]
                b1 = plsc.load_gather(bbuf, [o // CH])
                b2 = plsc.load_gather(bbuf, [onx // CH])
                e1 = e1buf[kk, sl] + b1
                e2 = e2buf[kk, sl] + b2
                cnt = (onx - o).astype(jnp.float32)
                whbuf[kk, sl] = (e2 - e1) / jnp.maximum(cnt, 1.0)
        pltpu.sync_copy(whbuf, wh_hbm.at[wid])

    return k(e, base, o3d, n3d)


def _segment_wh_sc(v, tok, opad, onext, NP):
    T, = tok.shape
    CH = T // _NW
    e, totals = _sc_scan(v, tok)
    base = jnp.concatenate(
        [jnp.zeros((1,), jnp.float32), jnp.cumsum(totals[:, 0])])
    base = jnp.concatenate([base, jnp.zeros((40 - _NW - 1,), jnp.float32)])
    K = NP // (_NW * 128)
    o3d = opad.reshape(_NW, K, 128)
    n3d = onext.reshape(_NW, K, 128)
    wh = _sc_boundaries(e, base, o3d, n3d, CH)
    return wh.reshape(NP)


def _factors(a_ref, whr_ref, whc_ref, j, BC):
    a0 = a_ref[0]
    a1 = a_ref[1]
    whc_all = whc_ref[0:1, :]
    D = jnp.max(whc_all * a1)
    whc_j = whc_ref[0:1, pl.ds(j * BC, BC)]
    d = whc_j * a1
    c1 = jnp.exp(d - D)
    c2 = jnp.exp(0.2 * (d - D))
    s = whr_ref[...] * a0
    sD = s + D
    m = jnp.where(sD > 0, sD, 0.2 * sD)
    r1 = jnp.exp(sD - m)
    r2 = jnp.exp(0.2 * sD - m)
    pos = d > (-s)
    return pos, r1, r2, c1, c2


def _attn_s1_body(a_ref, whr_ref, whc_ref, adj_ref, s1_ref,
                  P, Q, *, BC, NBJ, NN):
    j = pl.program_id(1)
    pos, r1, r2, c1, c2 = _factors(a_ref, whr_ref, whc_ref, j, BC)
    adj = adj_ref[...]
    col = lax.broadcasted_iota(jnp.int32, (1, BC), 1) + j * BC
    avalid = jnp.where(col < NN, adj, 0.0)
    ap = jnp.where(pos, avalid, 0.0)
    an = avalid - ap
    pP = jnp.sum(ap * c1, axis=1, keepdims=True)
    pQ = jnp.sum(an * c2, axis=1, keepdims=True)
    first = j == 0
    Pn = jnp.where(first, pP, P[...] + pP)
    Qn = jnp.where(first, pQ, Q[...] + pQ)
    P[...] = Pn
    Q[...] = Qn

    @pl.when(j == NBJ - 1)
    def _():
        s1_ref[...] = r1 * Pn + r2 * Qn


def _attn_out_body(a_ref, whr_ref, whc_ref, whcol_ref, s1_ref, adj_ref,
                   out_ref, ACC, *, BC, NBJ, NN):
    j = pl.program_id(1)
    pos, r1, r2, c1, c2 = _factors(a_ref, whr_ref, whc_ref, j, BC)
    adj = adj_ref[...]
    col = lax.broadcasted_iota(jnp.int32, (1, BC), 1) + j * BC
    avalid = jnp.where(col < NN, adj, 0.0)
    w = jnp.where(pos, r1 * c1, r2 * c2) * avalid
    S1 = s1_ref[...]
    rs = jnp.where(S1 > 0, 1.0 / S1, 0.0)
    uni = 1.0 / jnp.float32(NN)
    alpha = jnp.where(S1 > 0, w * rs, uni)
    partial = jnp.dot(alpha, whcol_ref[...],
                      preferred_element_type=jnp.float32)
    acc = jnp.where(j == 0, partial, ACC[...] + partial)
    ACC[...] = acc

    @pl.when(j == NBJ - 1)
    def _():
        out_ref[...] = acc


def _attention(adj, whr, whc, a2, interpret=False):
    NN = adj.shape[0]
    NP = whc.shape[1]
    BR, BC = 512, 2048
    NBI = (NN + BR - 1) // BR
    NBJ = (NN + BC - 1) // BC
    s1 = pl.pallas_call(
        functools.partial(_attn_s1_body, BC=BC, NBJ=NBJ, NN=NN),
        grid=(NBI, NBJ),
        in_specs=[pl.BlockSpec(memory_space=pltpu.SMEM),
                  pl.BlockSpec((BR, 1), lambda i, j: (i, 0)),
                  pl.BlockSpec((1, NP), lambda i, j: (0, 0)),
                  pl.BlockSpec((BR, BC), lambda i, j: (i, j))],
        out_specs=pl.BlockSpec((BR, 1), lambda i, j: (i, 0)),
        out_shape=jax.ShapeDtypeStruct((NN, 1), jnp.float32),
        scratch_shapes=[pltpu.VMEM((BR, 1), jnp.float32)] * 2,
        compiler_params=pltpu.CompilerParams(
            dimension_semantics=("arbitrary", "arbitrary")),
        interpret=interpret,
    )(a2, whr, whc, adj)
    whcol = whc.reshape(NP, 1)
    return pl.pallas_call(
        functools.partial(_attn_out_body, BC=BC, NBJ=NBJ, NN=NN),
        grid=(NBI, NBJ),
        in_specs=[pl.BlockSpec(memory_space=pltpu.SMEM),
                  pl.BlockSpec((BR, 1), lambda i, j: (i, 0)),
                  pl.BlockSpec((1, NP), lambda i, j: (0, 0)),
                  pl.BlockSpec((BC, 1), lambda i, j: (j, 0)),
                  pl.BlockSpec((BR, 1), lambda i, j: (i, 0)),
                  pl.BlockSpec((BR, BC), lambda i, j: (i, j))],
        out_specs=pl.BlockSpec((BR, 1), lambda i, j: (i, 0)),
        out_shape=jax.ShapeDtypeStruct((NN, 1), jnp.float32),
        scratch_shapes=[pltpu.VMEM((BR, 1), jnp.float32)],
        compiler_params=pltpu.CompilerParams(
            dimension_semantics=("arbitrary", "arbitrary")),
        interpret=interpret,
    )(a2, whr, whc, whcol, s1, adj)


_USE_SC = False


def kernel(inputs, offsets, adj_matrix, emb_table, W, a):
    T = inputs.shape[0]
    N = offsets.shape[0]
    NP = 12288

    tok = inputs.astype(jnp.int32)
    offs = offsets.astype(jnp.int32)
    opad = jnp.concatenate([offs, jnp.full((NP - N,), T, jnp.int32)])
    onext = jnp.concatenate([offs[1:], jnp.full((NP - N + 1,), T, jnp.int32)])

    v = _emb_matvec(emb_table, W).reshape(-1)
    if _USE_SC:
        wh = _segment_wh_sc(v, tok, opad, onext, NP)
    else:
        wh = _segment_wh_jnp(v, tok, opad, onext, NP)
    whr = wh[:N].reshape(N, 1)
    whc = wh[:10240].reshape(1, 10240)
    a2 = a.reshape(2).astype(jnp.float32)
    return _attention(adj_matrix, whr, whc, a2)


# speedup vs baseline: 39.7354x; 3.4539x over previous
import functools

import jax
import jax.numpy as jnp
from jax import lax
from jax.experimental import pallas as pl
from jax.experimental.pallas import tpu as pltpu
from jax.experimental.pallas import tpu_sc as plsc


def _matvec_body(emb_ref, w_ref, v_ref):
    v_ref[...] = jnp.dot(emb_ref[...], w_ref[...],
                         preferred_element_type=jnp.float32,
                         precision=lax.Precision.HIGHEST)


def _emb_matvec(emb, W, interpret=False):
    V, D = emb.shape
    BV = 5000
    return pl.pallas_call(
        _matvec_body,
        grid=(V // BV,),
        in_specs=[pl.BlockSpec((BV, D), lambda i: (i, 0)),
                  pl.BlockSpec((D, 1), lambda i: (0, 0))],
        out_specs=pl.BlockSpec((BV, 1), lambda i: (i, 0)),
        out_shape=jax.ShapeDtypeStruct((V, 1), jnp.float32),
        interpret=interpret,
    )(emb, W)


def _segment_wh_jnp(v, tok, opad, onext, NP):
    p = jnp.take(v, tok, axis=0)
    e = jnp.concatenate([jnp.zeros((1,), jnp.float32), jnp.cumsum(p)])
    e1 = jnp.take(e, opad)
    e2 = jnp.take(e, onext)
    cnt = (onext - opad).astype(jnp.float32)
    return (e2 - e1) / jnp.maximum(cnt, 1.0)


_NW = 32
_ROWS = 80
_CHT = _ROWS * 128
_TP = _NW * _CHT


def _sc_gather(vext, tok3d):
    mesh = plsc.VectorSubcoreMesh(core_axis_name="c", subcore_axis_name="s")

    @functools.partial(
        pl.kernel, mesh=mesh,
        out_type=jax.ShapeDtypeStruct((_NW, _ROWS, 128), jnp.float32),
        scratch_types=[pltpu.VMEM((_ROWS, 128), jnp.int32),
                       pltpu.VMEM((_ROWS, 128), jnp.float32),
                       pltpu.SemaphoreType.DMA],
    )
    def k(v_hbm, tok_hbm, p_hbm, ibuf, pbuf, sem):
        nc = 2
        wid = lax.axis_index("s") * nc + lax.axis_index("c")
        pltpu.sync_copy(tok_hbm.at[wid], ibuf)

        def gather_group(g, _):
            base_r = g * 8
            hs = [pltpu.async_copy(v_hbm.at[ibuf.at[base_r + i]],
                                   pbuf.at[base_r + i], sem)
                  for i in range(8)]
            for h in hs:
                h.wait()
            return 0

        lax.fori_loop(0, _ROWS // 8, gather_group, 0)
        pltpu.sync_copy(pbuf, p_hbm.at[wid])

    return k(vext, tok3d)


def _prefix_body(p_ref, e_ref):
    X = p_ref[...]
    R, C = X.shape
    li = lax.broadcasted_iota(jnp.int32, (C, C), 0)
    lj = lax.broadcasted_iota(jnp.int32, (C, C), 1)
    U = (li < lj).astype(jnp.float32)
    ones = jnp.ones((C, 1), jnp.float32)
    hp = lax.Precision.HIGHEST
    lane_excl = jnp.dot(X, U, precision=hp)
    rowsum = jnp.dot(X, ones, precision=hp)
    GS = 32
    NG = R // GS
    gi = lax.broadcasted_iota(jnp.int32, (R, R), 0)
    gj = lax.broadcasted_iota(jnp.int32, (R, R), 1)
    within = ((gi // GS == gj // GS) & (gj < gi)).astype(jnp.float32)
    sgi = lax.broadcasted_iota(jnp.int32, (NG, R), 0)
    sgj = lax.broadcasted_iota(jnp.int32, (NG, R), 1)
    Gsel = (sgi == sgj // GS).astype(jnp.float32)
    gsum = jnp.dot(Gsel, rowsum, precision=hp)
    mgi = lax.broadcasted_iota(jnp.int32, (NG, NG), 0)
    mgj = lax.broadcasted_iota(jnp.int32, (NG, NG), 1)
    Mg = (mgj < mgi).astype(jnp.float32)
    gpre = jnp.dot(Mg, gsum, precision=hp)
    pgi = lax.broadcasted_iota(jnp.int32, (R, NG), 0)
    pgj = lax.broadcasted_iota(jnp.int32, (R, NG), 1)
    Pexp = (pgj == pgi // GS).astype(jnp.float32)
    rowbase = (jnp.dot(Pexp, gpre, precision=hp)
               + jnp.dot(within, rowsum, precision=hp))
    e_ref[...] = rowbase + lane_excl


def _prefix_tc(p2d):
    R, C = p2d.shape
    return pl.pallas_call(
        _prefix_body,
        out_shape=jax.ShapeDtypeStruct((R, C), jnp.float32),
    )(p2d)


def _sc_boundaries(e, o3d, n3d):
    NW, K, L = o3d.shape
    mesh = plsc.VectorSubcoreMesh(core_axis_name="c", subcore_axis_name="s")

    @functools.partial(
        pl.kernel, mesh=mesh,
        out_type=jax.ShapeDtypeStruct((NW, K, L), jnp.float32),
        scratch_types=[pltpu.VMEM((K, L), jnp.int32),
                       pltpu.VMEM((K, L), jnp.int32),
                       pltpu.VMEM((K, L), jnp.float32),
                       pltpu.VMEM((K, L), jnp.float32),
                       pltpu.VMEM((K, L), jnp.float32),
                       pltpu.SemaphoreType.DMA],
    )
    def k(e_hbm, o_hbm, n_hbm, wh_hbm, obuf, nbuf, e1, e2, whbuf, sem):
        nc = 2
        wid = lax.axis_index("s") * nc + lax.axis_index("c")
        pltpu.sync_copy(o_hbm.at[wid], obuf)
        pltpu.sync_copy(n_hbm.at[wid], nbuf)
        hs = []
        for kk in range(K):
            hs.append(pltpu.async_copy(e_hbm.at[obuf.at[kk]], e1.at[kk], sem))
            hs.append(pltpu.async_copy(e_hbm.at[nbuf.at[kk]], e2.at[kk], sem))
        for h in hs:
            h.wait()
        for kk in range(K):
            for g in range(L // 16):
                sl = pl.ds(g * 16, 16)
                o = obuf[kk, sl]
                onx = nbuf[kk, sl]
                cnt = (onx - o).astype(jnp.float32)
                whbuf[kk, sl] = (e2[kk, sl] - e1[kk, sl]) / jnp.maximum(cnt, 1.0)
        pltpu.sync_copy(whbuf, wh_hbm.at[wid])

    return k(e, o3d, n3d)


def _segment_wh_sc(v, tok, opad, onext, NP):
    T, = tok.shape
    V, = v.shape
    vext = jnp.concatenate([v, jnp.zeros((16,), jnp.float32)])
    tok3d = jnp.concatenate(
        [tok, jnp.full((_TP - T,), V, jnp.int32)]).reshape(_NW, _ROWS, 128)
    p = _sc_gather(vext, tok3d)
    e2d = _prefix_tc(p.reshape(640, 512))
    e = e2d.reshape(_TP)
    K = NP // (_NW * 128)
    o3d = opad.reshape(_NW, K, 128)
    n3d = onext.reshape(_NW, K, 128)
    wh = _sc_boundaries(e, o3d, n3d)
    return wh.reshape(NP)


def _factors(a_ref, whr_ref, whc_ref, j, BC):
    a0 = a_ref[0]
    a1 = a_ref[1]
    whc_all = whc_ref[0:1, :]
    D = jnp.max(whc_all * a1)
    whc_j = whc_ref[0:1, pl.ds(j * BC, BC)]
    d = whc_j * a1
    c1 = jnp.exp(d - D)
    c2 = jnp.exp(0.2 * (d - D))
    s = whr_ref[...] * a0
    sD = s + D
    m = jnp.where(sD > 0, sD, 0.2 * sD)
    r1 = jnp.exp(sD - m)
    r2 = jnp.exp(0.2 * sD - m)
    pos = d > (-s)
    return pos, r1, r2, c1, c2


def _attn_s1_body(a_ref, whr_ref, whc_ref, adj_ref, s1_ref,
                  P, Q, *, BC, NBJ, NN):
    j = pl.program_id(1)
    pos, r1, r2, c1, c2 = _factors(a_ref, whr_ref, whc_ref, j, BC)
    adj = adj_ref[...]
    col = lax.broadcasted_iota(jnp.int32, (1, BC), 1) + j * BC
    avalid = jnp.where(col < NN, adj, 0.0)
    ap = jnp.where(pos, avalid, 0.0)
    an = avalid - ap
    pP = jnp.sum(ap * c1, axis=1, keepdims=True)
    pQ = jnp.sum(an * c2, axis=1, keepdims=True)
    first = j == 0
    Pn = jnp.where(first, pP, P[...] + pP)
    Qn = jnp.where(first, pQ, Q[...] + pQ)
    P[...] = Pn
    Q[...] = Qn

    @pl.when(j == NBJ - 1)
    def _():
        s1_ref[...] = r1 * Pn + r2 * Qn


def _attn_out_body(a_ref, whr_ref, whc_ref, whcol_ref, s1_ref, adj_ref,
                   out_ref, ACC, *, BC, NBJ, NN):
    j = pl.program_id(1)
    pos, r1, r2, c1, c2 = _factors(a_ref, whr_ref, whc_ref, j, BC)
    adj = adj_ref[...]
    col = lax.broadcasted_iota(jnp.int32, (1, BC), 1) + j * BC
    avalid = jnp.where(col < NN, adj, 0.0)
    w = jnp.where(pos, r1 * c1, r2 * c2) * avalid
    S1 = s1_ref[...]
    rs = jnp.where(S1 > 0, 1.0 / S1, 0.0)
    uni = 1.0 / jnp.float32(NN)
    alpha = jnp.where(S1 > 0, w * rs, uni)
    partial = jnp.dot(alpha, whcol_ref[...],
                      preferred_element_type=jnp.float32)
    acc = jnp.where(j == 0, partial, ACC[...] + partial)
    ACC[...] = acc

    @pl.when(j == NBJ - 1)
    def _():
        out_ref[...] = acc


def _attention(adj, whr, whc, a2, interpret=False):
    NN = adj.shape[0]
    NP = whc.shape[1]
    BR, BC = 512, 2048
    NBI = (NN + BR - 1) // BR
    NBJ = (NN + BC - 1) // BC
    s1 = pl.pallas_call(
        functools.partial(_attn_s1_body, BC=BC, NBJ=NBJ, NN=NN),
        grid=(NBI, NBJ),
        in_specs=[pl.BlockSpec(memory_space=pltpu.SMEM),
                  pl.BlockSpec((BR, 1), lambda i, j: (i, 0)),
                  pl.BlockSpec((1, NP), lambda i, j: (0, 0)),
                  pl.BlockSpec((BR, BC), lambda i, j: (i, j))],
        out_specs=pl.BlockSpec((BR, 1), lambda i, j: (i, 0)),
        out_shape=jax.ShapeDtypeStruct((NN, 1), jnp.float32),
        scratch_shapes=[pltpu.VMEM((BR, 1), jnp.float32)] * 2,
        compiler_params=pltpu.CompilerParams(
            dimension_semantics=("arbitrary", "arbitrary")),
        interpret=interpret,
    )(a2, whr, whc, adj)
    whcol = whc.reshape(NP, 1)
    return pl.pallas_call(
        functools.partial(_attn_out_body, BC=BC, NBJ=NBJ, NN=NN),
        grid=(NBI, NBJ),
        in_specs=[pl.BlockSpec(memory_space=pltpu.SMEM),
                  pl.BlockSpec((BR, 1), lambda i, j: (i, 0)),
                  pl.BlockSpec((1, NP), lambda i, j: (0, 0)),
                  pl.BlockSpec((BC, 1), lambda i, j: (j, 0)),
                  pl.BlockSpec((BR, 1), lambda i, j: (i, 0)),
                  pl.BlockSpec((BR, BC), lambda i, j: (i, j))],
        out_specs=pl.BlockSpec((BR, 1), lambda i, j: (i, 0)),
        out_shape=jax.ShapeDtypeStruct((NN, 1), jnp.float32),
        scratch_shapes=[pltpu.VMEM((BR, 1), jnp.float32)],
        compiler_params=pltpu.CompilerParams(
            dimension_semantics=("arbitrary", "arbitrary")),
        interpret=interpret,
    )(a2, whr, whc, whcol, s1, adj)


_USE_SC = True


def kernel(inputs, offsets, adj_matrix, emb_table, W, a):
    T = inputs.shape[0]
    N = offsets.shape[0]
    NP = 12288

    tok = inputs.astype(jnp.int32)
    offs = offsets.astype(jnp.int32)
    opad = jnp.concatenate([offs, jnp.full((NP - N,), T, jnp.int32)])
    onext = jnp.concatenate([offs[1:], jnp.full((NP - N + 1,), T, jnp.int32)])

    v = _emb_matvec(emb_table, W).reshape(-1)
    if _USE_SC:
        wh = _segment_wh_sc(v, tok, opad, onext, NP)
    else:
        wh = _segment_wh_jnp(v, tok, opad, onext, NP)
    whr = wh[:N].reshape(N, 1)
    whc = wh[:10240].reshape(1, 10240)
    a2 = a.reshape(2).astype(jnp.float32)
    return _attention(adj_matrix, whr, whc, a2)


# speedup vs baseline: 42.5094x; 1.0698x over previous
import functools

import jax
import jax.numpy as jnp
from jax import lax
from jax.experimental import pallas as pl
from jax.experimental.pallas import tpu as pltpu
from jax.experimental.pallas import tpu_sc as plsc


def _matvec_body(emb_ref, w_ref, v_ref):
    v_ref[...] = jnp.sum(emb_ref[...] * w_ref[...], axis=1, keepdims=True)


def _emb_matvec(emb, W, interpret=False):
    V, D = emb.shape
    BV = 5000
    return pl.pallas_call(
        _matvec_body,
        grid=(V // BV,),
        in_specs=[pl.BlockSpec((BV, D), lambda i: (i, 0)),
                  pl.BlockSpec((1, D), lambda i: (0, 0))],
        out_specs=pl.BlockSpec((BV, 1), lambda i: (i, 0)),
        out_shape=jax.ShapeDtypeStruct((V, 1), jnp.float32),
        interpret=interpret,
    )(emb, W.reshape(1, D))


def _segment_wh_jnp(v, tok, opad, onext, NP):
    p = jnp.take(v, tok, axis=0)
    e = jnp.concatenate([jnp.zeros((1,), jnp.float32), jnp.cumsum(p)])
    e1 = jnp.take(e, opad)
    e2 = jnp.take(e, onext)
    cnt = (onext - opad).astype(jnp.float32)
    return (e2 - e1) / jnp.maximum(cnt, 1.0)


_NW = 32
_ROWS = 80
_CHT = _ROWS * 128
_TP = _NW * _CHT


def _sc_gather(vext, tok3d):
    mesh = plsc.VectorSubcoreMesh(core_axis_name="c", subcore_axis_name="s")

    @functools.partial(
        pl.kernel, mesh=mesh,
        out_type=jax.ShapeDtypeStruct((_NW, _ROWS, 128), jnp.float32),
        scratch_types=[pltpu.VMEM((_ROWS, 128), jnp.int32),
                       pltpu.VMEM((_ROWS, 128), jnp.float32),
                       pltpu.SemaphoreType.DMA],
    )
    def k(v_hbm, tok_hbm, p_hbm, ibuf, pbuf, sem):
        nc = 2
        wid = lax.axis_index("s") * nc + lax.axis_index("c")
        pltpu.sync_copy(tok_hbm.at[wid], ibuf)

        def gather_group(g, _):
            base_r = g * 16
            hs = [pltpu.async_copy(v_hbm.at[ibuf.at[base_r + i]],
                                   pbuf.at[base_r + i], sem)
                  for i in range(16)]
            for h in hs:
                h.wait()
            return 0

        lax.fori_loop(0, _ROWS // 16, gather_group, 0)
        pltpu.sync_copy(pbuf, p_hbm.at[wid])

    return k(vext, tok3d)


def _prefix_body(p_ref, e_ref):
    X = p_ref[...]
    R, C = X.shape
    li = lax.broadcasted_iota(jnp.int32, (C, C), 0)
    lj = lax.broadcasted_iota(jnp.int32, (C, C), 1)
    U = (li < lj).astype(jnp.float32)
    ones = jnp.ones((C, 1), jnp.float32)
    hp = lax.Precision.HIGHEST
    lane_excl = jnp.dot(X, U, precision=hp)
    rowsum = jnp.dot(X, ones, precision=hp)
    GS = 32
    NG = R // GS
    gi = lax.broadcasted_iota(jnp.int32, (R, R), 0)
    gj = lax.broadcasted_iota(jnp.int32, (R, R), 1)
    within = ((gi // GS == gj // GS) & (gj < gi)).astype(jnp.float32)
    sgi = lax.broadcasted_iota(jnp.int32, (NG, R), 0)
    sgj = lax.broadcasted_iota(jnp.int32, (NG, R), 1)
    Gsel = (sgi == sgj // GS).astype(jnp.float32)
    gsum = jnp.dot(Gsel, rowsum, precision=hp)
    mgi = lax.broadcasted_iota(jnp.int32, (NG, NG), 0)
    mgj = lax.broadcasted_iota(jnp.int32, (NG, NG), 1)
    Mg = (mgj < mgi).astype(jnp.float32)
    gpre = jnp.dot(Mg, gsum, precision=hp)
    pgi = lax.broadcasted_iota(jnp.int32, (R, NG), 0)
    pgj = lax.broadcasted_iota(jnp.int32, (R, NG), 1)
    Pexp = (pgj == pgi // GS).astype(jnp.float32)
    rowbase = (jnp.dot(Pexp, gpre, precision=hp)
               + jnp.dot(within, rowsum, precision=hp))
    e_ref[...] = rowbase + lane_excl


def _prefix_tc(p2d):
    R, C = p2d.shape
    return pl.pallas_call(
        _prefix_body,
        out_shape=jax.ShapeDtypeStruct((R, C), jnp.float32),
    )(p2d)


def _sc_boundaries(e, o3d, n3d):
    NW, K, L = o3d.shape
    mesh = plsc.VectorSubcoreMesh(core_axis_name="c", subcore_axis_name="s")

    @functools.partial(
        pl.kernel, mesh=mesh,
        out_type=jax.ShapeDtypeStruct((NW, K, L), jnp.float32),
        scratch_types=[pltpu.VMEM((K, L), jnp.int32),
                       pltpu.VMEM((K, L), jnp.int32),
                       pltpu.VMEM((K, L), jnp.float32),
                       pltpu.VMEM((K, L), jnp.float32),
                       pltpu.VMEM((K, L), jnp.float32),
                       pltpu.SemaphoreType.DMA],
    )
    def k(e_hbm, o_hbm, n_hbm, wh_hbm, obuf, nbuf, e1, e2, whbuf, sem):
        nc = 2
        wid = lax.axis_index("s") * nc + lax.axis_index("c")
        pltpu.sync_copy(o_hbm.at[wid], obuf)
        pltpu.sync_copy(n_hbm.at[wid], nbuf)
        hs = []
        for kk in range(K):
            hs.append(pltpu.async_copy(e_hbm.at[obuf.at[kk]], e1.at[kk], sem))
            hs.append(pltpu.async_copy(e_hbm.at[nbuf.at[kk]], e2.at[kk], sem))
        for h in hs:
            h.wait()
        for kk in range(K):
            for g in range(L // 16):
                sl = pl.ds(g * 16, 16)
                o = obuf[kk, sl]
                onx = nbuf[kk, sl]
                cnt = (onx - o).astype(jnp.float32)
                whbuf[kk, sl] = (e2[kk, sl] - e1[kk, sl]) / jnp.maximum(cnt, 1.0)
        pltpu.sync_copy(whbuf, wh_hbm.at[wid])

    return k(e, o3d, n3d)


def _segment_wh_sc(v, tok, opad, onext, NP):
    T, = tok.shape
    V, = v.shape
    vext = jnp.concatenate([v, jnp.zeros((16,), jnp.float32)])
    tok3d = jnp.concatenate(
        [tok, jnp.full((_TP - T,), V, jnp.int32)]).reshape(_NW, _ROWS, 128)
    p = _sc_gather(vext, tok3d)
    e2d = _prefix_tc(p.reshape(640, 512))
    e = e2d.reshape(_TP)
    K = NP // (_NW * 128)
    o3d = opad.reshape(_NW, K, 128)
    n3d = onext.reshape(_NW, K, 128)
    wh = _sc_boundaries(e, o3d, n3d)
    return wh.reshape(NP)


def _factors(a_ref, whr_ref, whc_ref, j, BC):
    a0 = a_ref[0]
    a1 = a_ref[1]
    whc_all = whc_ref[0:1, :]
    D = jnp.max(whc_all * a1)
    whc_j = whc_ref[0:1, pl.ds(j * BC, BC)]
    d = whc_j * a1
    c1 = jnp.exp(d - D)
    c2 = jnp.exp(0.2 * (d - D))
    s = whr_ref[...] * a0
    sD = s + D
    m = jnp.where(sD > 0, sD, 0.2 * sD)
    r1 = jnp.exp(sD - m)
    r2 = jnp.exp(0.2 * sD - m)
    pos = d > (-s)
    return pos, r1, r2, c1, c2


def _attn_s1_body(a_ref, whr_ref, whc_ref, adj_ref, s1_ref,
                  P, Q, *, BC, NBJ, NN):
    j = pl.program_id(1)
    pos, r1, r2, c1, c2 = _factors(a_ref, whr_ref, whc_ref, j, BC)
    adj = adj_ref[...]
    col = lax.broadcasted_iota(jnp.int32, (1, BC), 1) + j * BC
    avalid = jnp.where(col < NN, adj, 0.0)
    ap = jnp.where(pos, avalid, 0.0)
    an = avalid - ap
    pP = jnp.sum(ap * c1, axis=1, keepdims=True)
    pQ = jnp.sum(an * c2, axis=1, keepdims=True)
    first = j == 0
    Pn = jnp.where(first, pP, P[...] + pP)
    Qn = jnp.where(first, pQ, Q[...] + pQ)
    P[...] = Pn
    Q[...] = Qn

    @pl.when(j == NBJ - 1)
    def _():
        s1_ref[...] = r1 * Pn + r2 * Qn


def _attn_out_body(a_ref, whr_ref, whc_ref, whcol_ref, s1_ref, adj_ref,
                   out_ref, ACC, *, BC, NBJ, NN):
    j = pl.program_id(1)
    pos, r1, r2, c1, c2 = _factors(a_ref, whr_ref, whc_ref, j, BC)
    adj = adj_ref[...]
    col = lax.broadcasted_iota(jnp.int32, (1, BC), 1) + j * BC
    avalid = jnp.where(col < NN, adj, 0.0)
    w = jnp.where(pos, r1 * c1, r2 * c2) * avalid
    S1 = s1_ref[...]
    rs = jnp.where(S1 > 0, 1.0 / S1, 0.0)
    uni = 1.0 / jnp.float32(NN)
    alpha = jnp.where(S1 > 0, w * rs, uni)
    partial = jnp.dot(alpha, whcol_ref[...],
                      preferred_element_type=jnp.float32)
    acc = jnp.where(j == 0, partial, ACC[...] + partial)
    ACC[...] = acc

    @pl.when(j == NBJ - 1)
    def _():
        out_ref[...] = acc


def _attention(adj, whr, whc, a2, interpret=False):
    NN = adj.shape[0]
    NP = whc.shape[1]
    BR, BC = 512, 2048
    NBI = (NN + BR - 1) // BR
    NBJ = (NN + BC - 1) // BC
    s1 = pl.pallas_call(
        functools.partial(_attn_s1_body, BC=BC, NBJ=NBJ, NN=NN),
        grid=(NBI, NBJ),
        in_specs=[pl.BlockSpec(memory_space=pltpu.SMEM),
                  pl.BlockSpec((BR, 1), lambda i, j: (i, 0)),
                  pl.BlockSpec((1, NP), lambda i, j: (0, 0)),
                  pl.BlockSpec((BR, BC), lambda i, j: (i, j))],
        out_specs=pl.BlockSpec((BR, 1), lambda i, j: (i, 0)),
        out_shape=jax.ShapeDtypeStruct((NN, 1), jnp.float32),
        scratch_shapes=[pltpu.VMEM((BR, 1), jnp.float32)] * 2,
        compiler_params=pltpu.CompilerParams(
            dimension_semantics=("arbitrary", "arbitrary")),
        interpret=interpret,
    )(a2, whr, whc, adj)
    whcol = whc.reshape(NP, 1)
    return pl.pallas_call(
        functools.partial(_attn_out_body, BC=BC, NBJ=NBJ, NN=NN),
        grid=(NBI, NBJ),
        in_specs=[pl.BlockSpec(memory_space=pltpu.SMEM),
                  pl.BlockSpec((BR, 1), lambda i, j: (i, 0)),
                  pl.BlockSpec((1, NP), lambda i, j: (0, 0)),
                  pl.BlockSpec((BC, 1), lambda i, j: (j, 0)),
                  pl.BlockSpec((BR, 1), lambda i, j: (i, 0)),
                  pl.BlockSpec((BR, BC), lambda i, j: (i, j))],
        out_specs=pl.BlockSpec((BR, 1), lambda i, j: (i, 0)),
        out_shape=jax.ShapeDtypeStruct((NN, 1), jnp.float32),
        scratch_shapes=[pltpu.VMEM((BR, 1), jnp.float32)],
        compiler_params=pltpu.CompilerParams(
            dimension_semantics=("arbitrary", "arbitrary")),
        interpret=interpret,
    )(a2, whr, whc, whcol, s1, adj)


_USE_SC = True


def kernel(inputs, offsets, adj_matrix, emb_table, W, a):
    T = inputs.shape[0]
    N = offsets.shape[0]
    NP = 12288

    tok = inputs.astype(jnp.int32)
    offs = offsets.astype(jnp.int32)
    opad = jnp.concatenate([offs, jnp.full((NP - N,), T, jnp.int32)])
    onext = jnp.concatenate([offs[1:], jnp.full((NP - N + 1,), T, jnp.int32)])

    v = _emb_matvec(emb_table, W).reshape(-1)
    if _USE_SC:
        wh = _segment_wh_sc(v, tok, opad, onext, NP)
    else:
        wh = _segment_wh_jnp(v, tok, opad, onext, NP)
    whr = wh[:N].reshape(N, 1)
    whc = wh[:10240].reshape(1, 10240)
    a2 = a.reshape(2).astype(jnp.float32)
    return _attention(adj_matrix, whr, whc, a2)
